# Initial kernel scaffold; baseline (speedup 1.0000x reference)
import jax
import jax.numpy as jnp
from jax.experimental import pallas as pl


def kernel(xyz, fvec, targets, coords_world_xyz):
    raise NotImplementedError("write your pallas kernel here")


# speedup vs baseline: 5.3878x; 5.3878x over previous
import functools

import jax
import jax.numpy as jnp
from jax import lax
from jax.experimental import pallas as pl

K = 16
GROUPS = 8
BLK = 128


def _d2_body(q_ref, x_ref, d2_ref, bm_ref):
    q = q_ref[0]
    x = x_ref[0]
    qn = jnp.sum(q * q, axis=1)
    pn = jnp.sum(x * x, axis=0)
    cross = jnp.dot(q, x, preferred_element_type=jnp.float32)
    d2 = qn[:, None] + pn[None, :] - 2.0 * cross
    d2_ref[0] = d2
    tm, tn = d2.shape
    bm_ref[0, 0] = jnp.min(d2.reshape(tm, tn // BLK, BLK), axis=-1)


def _d2_and_blockmins(coords, xyzT):
    B, M, _ = coords.shape
    N = xyzT.shape[2]
    TM, TN = 256, 2048
    grid = (B, M // TM, N // TN)
    return pl.pallas_call(
        _d2_body,
        grid=grid,
        in_specs=[
            pl.BlockSpec((1, TM, 3), lambda b, i, j: (b, i, 0)),
            pl.BlockSpec((1, 3, TN), lambda b, i, j: (b, 0, j)),
        ],
        out_specs=[
            pl.BlockSpec((1, TM, TN), lambda b, i, j: (b, i, j)),
            pl.BlockSpec((1, 1, TM, TN // BLK), lambda b, i, j: (b, j, i, 0)),
        ],
        out_shape=[
            jax.ShapeDtypeStruct((B, M, N), jnp.float32),
            jax.ShapeDtypeStruct((B, N // TN, M, TN // BLK), jnp.float32),
        ],
    )(coords, xyzT)


def kernel(xyz, fvec, targets, coords_world_xyz):
    B, N, C = fvec.shape
    M = targets.shape[1]

    xyzT = jnp.transpose(xyz, (0, 2, 1))
    d2, bmins4 = _d2_and_blockmins(coords_world_xyz, xyzT)
    bmins = jnp.transpose(bmins4, (0, 2, 1, 3)).reshape(B, M, N // BLK)

    bidx = lax.top_k(-bmins, K)[1]
    bidx = jnp.sort(bidx, axis=-1)
    cand = jnp.take_along_axis(
        d2.reshape(B, M, N // BLK, BLK), bidx[..., None], axis=2
    )
    flat = cand.reshape(B, M, K * BLK)
    _, pos = lax.top_k(-flat, K)
    gidx = jnp.take_along_axis(bidx, pos // BLK, axis=-1) * BLK + pos % BLK

    batch_idx = jnp.arange(B)[:, None, None]
    neighbor_xyz = xyz[batch_idx, gidx]
    neighbor_fvec = fvec[batch_idx, gidx]

    tg = targets.reshape(B, M, GROUPS, -1)
    nfg = neighbor_fvec.reshape(B, M, K, GROUPS, -1)
    corrs = jnp.einsum('BMGc,BMKGc->BMKG', tg, nfg)
    corrs = corrs / (float(C) / GROUPS) ** 0.5

    offset = neighbor_xyz - coords_world_xyz[:, :, None, :]
    return jnp.concatenate([corrs, offset, neighbor_xyz], axis=-1)


# speedup vs baseline: 29.7567x; 5.5230x over previous
import functools

import jax
import jax.numpy as jnp
import numpy as np
from jax import lax
from jax.experimental import pallas as pl
from jax.experimental.pallas import tpu as pltpu
from jax.experimental.pallas import tpu_sc as plsc

K = 16
GROUPS = 8
BLK = 128
NB = 128
NC, NS = 2, 16
NW = NC * NS
QPW = 128
CQ = 8
ROWS = CQ * K
NCHUNK = QPW // CQ
INF = np.float32(np.inf)


def _d2_body(q_ref, x_ref, d2_ref, bm_ref):
    q = q_ref[0]
    x = x_ref[0]
    qn = jnp.sum(q * q, axis=1)
    pn = jnp.sum(x * x, axis=0)
    cross = jnp.dot(q, x, preferred_element_type=jnp.float32)
    d2 = qn[:, None] + pn[None, :] - 2.0 * cross
    d2_ref[0] = d2
    tm, tn = d2.shape
    bm_ref[0, 0] = jnp.min(d2.reshape(tm, tn // BLK, BLK), axis=-1)


def _d2_and_blockmins(coords, xyzT):
    B, M, _ = coords.shape
    N = xyzT.shape[2]
    TM, TN = 256, 2048
    return pl.pallas_call(
        _d2_body,
        grid=(B, M // TM, N // TN),
        in_specs=[
            pl.BlockSpec((1, TM, 3), lambda b, i, j: (b, i, 0)),
            pl.BlockSpec((1, 3, TN), lambda b, i, j: (b, 0, j)),
        ],
        out_specs=[
            pl.BlockSpec((1, TM, TN), lambda b, i, j: (b, i, j)),
            pl.BlockSpec((1, 1, TM, TN // BLK), lambda b, i, j: (b, j, i, 0)),
        ],
        out_shape=[
            jax.ShapeDtypeStruct((B, M, N), jnp.float32),
            jax.ShapeDtypeStruct((B, N // TN, M, TN // BLK), jnp.float32),
        ],
    )(coords, xyzT)


def _merge16(ak, av, bk, bv):
    rbk = lax.rev(bk, (0,))
    rbv = lax.rev(bv, (0,))
    m = ak <= rbk
    mk = jnp.where(m, ak, rbk)
    mv = jnp.where(m, av, rbv)
    sk, sv = plsc.sort_key_val(mk, mv)
    return sk, sv


def _sc_select_gather(bm, d2r, xyzpad, coordpad, fvecf, BM, N):
    mesh = plsc.VectorSubcoreMesh(core_axis_name="c", subcore_axis_name="s")

    @functools.partial(
        pl.kernel,
        out_type=[
            jax.ShapeDtypeStruct((BM * K, 256), jnp.float32),
            jax.ShapeDtypeStruct((BM * K, 16), jnp.float32),
        ],
        mesh=mesh,
        compiler_params=pltpu.CompilerParams(
            needs_layout_passes=False, use_tc_tiling_on_sc=False),
        scratch_types=[
            pltpu.VMEM((QPW, NB), jnp.float32),
            pltpu.VMEM((QPW, 16), jnp.float32),
            pltpu.VMEM((QPW * K,), jnp.int32),
            pltpu.SMEM((QPW,), jnp.float32),
            pltpu.VMEM((2, ROWS, BLK), jnp.float32),
            pltpu.VMEM((2, ROWS), jnp.int32),
            pltpu.VMEM((2, ROWS, 256), jnp.float32),
            pltpu.VMEM((2, ROWS, 16), jnp.float32),
            pltpu.VMEM((2, ROWS, 16), jnp.float32),
            pltpu.SemaphoreType.DMA((2,)),
            pltpu.SemaphoreType.DMA,
            pltpu.SemaphoreType.DMA,
        ],
    )
    def body(bm_hbm, d2r_hbm, xyz_hbm, cp_hbm, fv_hbm, gath_hbm, xout_hbm,
             bmv, cpv, rowflat, thrv, candv, fidxv, fbufv, xgv, xov,
             cand_sem, fg_sem, xg_sem):
        wid = lax.axis_index("s") * NC + lax.axis_index("c")
        qbase = wid * QPW
        nbase = (qbase // 2048) * N
        iota = lax.iota(jnp.int32, 16)
        shift_idx = jnp.where(iota < 3, iota,
                              jnp.where(iota < 6, iota - 3, 15))

        pltpu.sync_copy(bm_hbm.at[pl.ds(qbase, QPW)], bmv)
        pltpu.sync_copy(cp_hbm.at[pl.ds(qbase, QPW)], cpv)

        def p1(q, _):
            ks, vs = [], []
            for r in range(8):
                k = bmv[q, pl.ds(r * 16, 16)]
                k, v = plsc.sort_key_val(k, iota + r * 16)
                ks.append(k)
                vs.append(v)
            while len(ks) > 1:
                nk, nv = [], []
                for i in range(0, len(ks), 2):
                    a, b = _merge16(ks[i], vs[i], ks[i + 1], vs[i + 1])
                    nk.append(a)
                    nv.append(b)
                ks, vs = nk, nv
            thrv[q] = jnp.max(ks[0])
            sid, _ = plsc.sort_key_val(vs[0], vs[0])
            rowflat[pl.ds(q * 16, 16)] = sid + (qbase + q) * NB
            return 0

        lax.fori_loop(0, QPW, p1, 0)

        def cand_fetch(chunk, slot):
            return pltpu.async_copy(
                d2r_hbm.at[rowflat.at[pl.ds(chunk * ROWS, ROWS)]],
                candv.at[slot], cand_sem.at[slot])

        cand_fetch(0, 0)
        cand_fetch(1, 1)

        def chunk_body(p, _):
            for s in range(2):
                c = p * 2 + s
                pltpu.make_async_copy(
                    d2r_hbm.at[pl.ds(0, ROWS)], candv.at[s],
                    cand_sem.at[s]).wait()

                def q_body(q2, _):
                    lq = c * CQ + q2
                    theta = thrv[lq]
                    qg128 = (qbase + lq) * NB

                    def bi_body(bi, cur):
                        ck, cv = cur
                        blkb = plsc.load_gather(
                            rowflat,
                            [jnp.full((16,), lq * 16 + bi, jnp.int32)])
                        row = q2 * 16 + bi
                        for part in range(8):
                            v = candv[s, row, pl.ds(part * 16, 16)]
                            msk = v <= theta
                            vidx = (blkb - qg128) * BLK + part * 16 + iota

                            def do(ck, cv, v, vidx, msk):
                                sk, sv = plsc.sort_key_val(
                                    jnp.where(msk, v, INF), vidx)
                                return _merge16(ck, cv, sk, sv)

                            ck, cv = lax.cond(
                                jnp.any(msk), do,
                                lambda ck, cv, v, vidx, msk: (ck, cv),
                                ck, cv, v, vidx, msk)
                        return ck, cv

                    _, fin_v = lax.fori_loop(
                        0, 16, bi_body,
                        (jnp.full((16,), INF, jnp.float32),
                         jnp.zeros((16,), jnp.int32)))
                    fidxv[s, pl.ds(q2 * 16, 16)] = fin_v + nbase
                    return 0

                lax.fori_loop(0, CQ, q_body, 0)

                pltpu.async_copy(fv_hbm.at[fidxv.at[s]], fbufv.at[s], fg_sem)
                pltpu.async_copy(xyz_hbm.at[fidxv.at[s]], xgv.at[s], xg_sem)
                pltpu.make_async_copy(
                    xyz_hbm.at[pl.ds(0, ROWS)], xgv.at[s], xg_sem).wait()

                def x_body(q2, _):
                    qrow = cpv[c * CQ + q2]
                    for k in range(16):
                        row = q2 * 16 + k
                        sh = plsc.load_gather(
                            xgv, [jnp.full((16,), s, jnp.int32),
                                  jnp.full((16,), row, jnp.int32), shift_idx])
                        xov[s, row] = sh - qrow
                    return 0

                lax.fori_loop(0, CQ, x_body, 0)

                obase = (qbase + c * CQ) * K
                pltpu.make_async_copy(
                    fv_hbm.at[pl.ds(0, ROWS)], fbufv.at[s], fg_sem).wait()
                pltpu.sync_copy(fbufv.at[s], gath_hbm.at[pl.ds(obase, ROWS)])
                pltpu.sync_copy(xov.at[s], xout_hbm.at[pl.ds(obase, ROWS)])

                cand_fetch(jnp.minimum(c + 2, NCHUNK - 1), s)
            return 0

        lax.fori_loop(0, NCHUNK // 2, chunk_body, 0)
        for s in range(2):
            pltpu.make_async_copy(
                d2r_hbm.at[pl.ds(0, ROWS)], candv.at[s], cand_sem.at[s]).wait()

    return body(bm, d2r, xyzpad, coordpad, fvecf)


def _corr_body(g_ref, t_ref, p_ref, o_ref):
    g = g_ref[...]
    t = t_ref[...]
    tq = t.shape[0]
    trep = jnp.broadcast_to(t[:, None, :], (tq, K, 256)).reshape(tq * K, 256)
    z = g * trep
    o_ref[...] = jnp.dot(z, p_ref[...], preferred_element_type=jnp.float32) \
        / np.float32((256.0 / GROUPS) ** 0.5)


def _corr(gath, targets_flat, pmask):
    R = gath.shape[0]
    TQ = 128
    return pl.pallas_call(
        _corr_body,
        grid=(R // (TQ * K),),
        in_specs=[
            pl.BlockSpec((TQ * K, 256), lambda i: (i, 0)),
            pl.BlockSpec((TQ, 256), lambda i: (i, 0)),
            pl.BlockSpec((256, GROUPS), lambda i: (0, 0)),
        ],
        out_specs=pl.BlockSpec((TQ * K, GROUPS), lambda i: (i, 0)),
        out_shape=jax.ShapeDtypeStruct((R, GROUPS), jnp.float32),
    )(gath, targets_flat, pmask)


def kernel(xyz, fvec, targets, coords_world_xyz):
    B, N, C = fvec.shape
    M = targets.shape[1]
    BM = B * M

    xyzT = jnp.transpose(xyz, (0, 2, 1))
    d2, bmins4 = _d2_and_blockmins(coords_world_xyz, xyzT)
    bm = jnp.transpose(bmins4, (0, 2, 1, 3)).reshape(BM, N // BLK)

    d2r = d2.reshape(BM * NB, BLK)
    xyzpad = jnp.concatenate(
        [xyz.reshape(B * N, 3),
         jnp.zeros((B * N, 13), jnp.float32)], axis=-1)
    coordpad = jnp.concatenate(
        [coords_world_xyz.reshape(BM, 3),
         jnp.zeros((BM, 13), jnp.float32)], axis=-1)
    fvecf = fvec.reshape(B * N, C)

    gath, xout = _sc_select_gather(bm, d2r, xyzpad, coordpad, fvecf, BM, N)

    pmask = jnp.asarray(
        (np.arange(256)[:, None] // (C // GROUPS)
         == np.arange(GROUPS)[None, :]).astype(np.float32))
    corrs = _corr(gath, targets.reshape(BM, C), pmask).reshape(B, M, K, GROUPS)

    xyzpart = xout.reshape(B, M, K, 16)[..., :6]
    return jnp.concatenate([corrs, xyzpart], axis=-1)


# speedup vs baseline: 30.7176x; 1.0323x over previous
import functools

import jax
import jax.numpy as jnp
import numpy as np
from jax import lax
from jax.experimental import pallas as pl
from jax.experimental.pallas import tpu as pltpu
from jax.experimental.pallas import tpu_sc as plsc

K = 16
GROUPS = 8
BLK = 128
NB = 128
NC, NS = 2, 16
NW = NC * NS
QPW = 128
CQ = 8
ROWS = CQ * K
NCHUNK = QPW // CQ
INF = np.float32(np.inf)


def _d2_body(q_ref, x_ref, d2_ref, bm_ref):
    q = q_ref[0]
    x = x_ref[0]
    qn = jnp.sum(q * q, axis=1)
    pn = jnp.sum(x * x, axis=0)
    cross = jnp.dot(q, x, preferred_element_type=jnp.float32)
    d2 = qn[:, None] + pn[None, :] - 2.0 * cross
    d2_ref[0] = d2
    tm, tn = d2.shape
    bm_ref[0, 0] = jnp.min(d2.reshape(tm, tn // BLK, BLK), axis=-1)


def _d2_and_blockmins(coords, xyzT):
    B, M, _ = coords.shape
    N = xyzT.shape[2]
    TM, TN = 256, 2048
    return pl.pallas_call(
        _d2_body,
        grid=(B, M // TM, N // TN),
        in_specs=[
            pl.BlockSpec((1, TM, 3), lambda b, i, j: (b, i, 0)),
            pl.BlockSpec((1, 3, TN), lambda b, i, j: (b, 0, j)),
        ],
        out_specs=[
            pl.BlockSpec((1, TM, TN), lambda b, i, j: (b, i, j)),
            pl.BlockSpec((1, 1, TM, TN // BLK), lambda b, i, j: (b, j, i, 0)),
        ],
        out_shape=[
            jax.ShapeDtypeStruct((B, M, N), jnp.float32),
            jax.ShapeDtypeStruct((B, N // TN, M, TN // BLK), jnp.float32),
        ],
    )(coords, xyzT)


def _merge16(ak, av, bk, bv):
    rbk = lax.rev(bk, (0,))
    rbv = lax.rev(bv, (0,))
    m = ak <= rbk
    mk = jnp.where(m, ak, rbk)
    mv = jnp.where(m, av, rbv)
    sk, sv = plsc.sort_key_val(mk, mv)
    return sk, sv


def _sc_select_gather(bm4, d2r, xyzpad, coordpad, fvecf, BM, N):
    mesh = plsc.VectorSubcoreMesh(core_axis_name="c", subcore_axis_name="s")

    @functools.partial(
        pl.kernel,
        out_type=[
            jax.ShapeDtypeStruct((BM * K, 256), jnp.float32),
            jax.ShapeDtypeStruct((BM * K, 16), jnp.float32),
        ],
        mesh=mesh,
        compiler_params=pltpu.CompilerParams(
            needs_layout_passes=False, use_tc_tiling_on_sc=False),
        scratch_types=[
            pltpu.VMEM((8, QPW, 16), jnp.float32),
            pltpu.VMEM((QPW, 16), jnp.float32),
            pltpu.VMEM((QPW * K,), jnp.int32),
            pltpu.SMEM((QPW,), jnp.float32),
            pltpu.VMEM((2, ROWS, BLK), jnp.float32),
            pltpu.VMEM((2, ROWS), jnp.int32),
            pltpu.VMEM((2, ROWS, 256), jnp.float32),
            pltpu.VMEM((2, ROWS, 16), jnp.float32),
            pltpu.VMEM((2, ROWS, 16), jnp.float32),
            pltpu.SemaphoreType.DMA((2,)),
            pltpu.SemaphoreType.DMA,
            pltpu.SemaphoreType.DMA,
        ],
    )
    def body(bm_hbm, d2r_hbm, xyz_hbm, cp_hbm, fv_hbm, gath_hbm, xout_hbm,
             bmv, cpv, rowflat, thrv, candv, fidxv, fbufv, xgv, xov,
             cand_sem, fg_sem, xg_sem):
        wid = lax.axis_index("s") * NC + lax.axis_index("c")
        qbase = wid * QPW
        b = qbase // 2048
        mq = qbase - b * 2048
        nbase = b * N
        iota = lax.iota(jnp.int32, 16)
        shift_idx = jnp.where(iota < 3, iota,
                              jnp.where(iota < 6, iota - 3, 15))

        pltpu.sync_copy(bm_hbm.at[b, :, pl.ds(mq, QPW), :], bmv)
        pltpu.sync_copy(cp_hbm.at[pl.ds(qbase, QPW)], cpv)

        def p1(q, _):
            ks, vs = [], []
            for r in range(8):
                k = bmv[r, q, pl.ds(0, 16)]
                k, v = plsc.sort_key_val(k, iota + r * 16)
                ks.append(k)
                vs.append(v)
            while len(ks) > 1:
                nk, nv = [], []
                for i in range(0, len(ks), 2):
                    a, b = _merge16(ks[i], vs[i], ks[i + 1], vs[i + 1])
                    nk.append(a)
                    nv.append(b)
                ks, vs = nk, nv
            thrv[q] = jnp.max(ks[0])
            sid, _ = plsc.sort_key_val(vs[0], vs[0])
            rowflat[pl.ds(q * 16, 16)] = sid + (qbase + q) * NB
            return 0

        lax.fori_loop(0, QPW, p1, 0)

        def cand_fetch(chunk, slot):
            return pltpu.async_copy(
                d2r_hbm.at[rowflat.at[pl.ds(chunk * ROWS, ROWS)]],
                candv.at[slot], cand_sem.at[slot])

        cand_fetch(0, 0)
        cand_fetch(1, 1)

        def chunk_body(p, _):
            for s in range(2):
                c = p * 2 + s
                pltpu.make_async_copy(
                    d2r_hbm.at[pl.ds(0, ROWS)], candv.at[s],
                    cand_sem.at[s]).wait()

                def q_body(q2, _):
                    lq = c * CQ + q2
                    theta = thrv[lq]
                    qg128 = (qbase + lq) * NB

                    def bi_body(bi, cur):
                        ck, cv = cur
                        blkb = plsc.load_gather(
                            rowflat,
                            [jnp.full((16,), lq * 16 + bi, jnp.int32)])
                        row = q2 * 16 + bi
                        for part in range(8):
                            v = candv[s, row, pl.ds(part * 16, 16)]
                            msk = v <= theta
                            vidx = (blkb - qg128) * BLK + part * 16 + iota

                            def do(ck, cv, v, vidx, msk):
                                sk, sv = plsc.sort_key_val(
                                    jnp.where(msk, v, INF), vidx)
                                return _merge16(ck, cv, sk, sv)

                            ck, cv = lax.cond(
                                jnp.any(msk), do,
                                lambda ck, cv, v, vidx, msk: (ck, cv),
                                ck, cv, v, vidx, msk)
                        return ck, cv

                    _, fin_v = lax.fori_loop(
                        0, 16, bi_body,
                        (jnp.full((16,), INF, jnp.float32),
                         jnp.zeros((16,), jnp.int32)))
                    fidxv[s, pl.ds(q2 * 16, 16)] = fin_v + nbase
                    return 0

                lax.fori_loop(0, CQ, q_body, 0)

                pltpu.async_copy(fv_hbm.at[fidxv.at[s]], fbufv.at[s], fg_sem)
                pltpu.async_copy(xyz_hbm.at[fidxv.at[s]], xgv.at[s], xg_sem)
                pltpu.make_async_copy(
                    xyz_hbm.at[pl.ds(0, ROWS)], xgv.at[s], xg_sem).wait()

                def x_body(q2, _):
                    qrow = cpv[c * CQ + q2]
                    for k in range(16):
                        row = q2 * 16 + k
                        sh = plsc.load_gather(
                            xgv, [jnp.full((16,), s, jnp.int32),
                                  jnp.full((16,), row, jnp.int32), shift_idx])
                        xov[s, row] = sh - qrow
                    return 0

                lax.fori_loop(0, CQ, x_body, 0)

                obase = (qbase + c * CQ) * K
                pltpu.make_async_copy(
                    fv_hbm.at[pl.ds(0, ROWS)], fbufv.at[s], fg_sem).wait()
                pltpu.sync_copy(fbufv.at[s], gath_hbm.at[pl.ds(obase, ROWS)])
                pltpu.sync_copy(xov.at[s], xout_hbm.at[pl.ds(obase, ROWS)])

                cand_fetch(jnp.minimum(c + 2, NCHUNK - 1), s)
            return 0

        lax.fori_loop(0, NCHUNK // 2, chunk_body, 0)
        for s in range(2):
            pltpu.make_async_copy(
                d2r_hbm.at[pl.ds(0, ROWS)], candv.at[s], cand_sem.at[s]).wait()

    return body(bm4, d2r, xyzpad, coordpad, fvecf)


def _corr_body(g_ref, t_ref, p_ref, x_ref, o_ref):
    g = g_ref[...]
    t = t_ref[...]
    tq = t.shape[0]
    trep = jnp.broadcast_to(t[:, None, :], (tq, K, 256)).reshape(tq * K, 256)
    z = g * trep
    corr = jnp.dot(z, p_ref[...], preferred_element_type=jnp.float32) \
        / np.float32((256.0 / GROUPS) ** 0.5)
    o_ref[...] = jnp.concatenate([corr, x_ref[...][:, :6]], axis=1)


def _corr(gath, targets_flat, pmask, xout):
    R = gath.shape[0]
    TQ = 128
    return pl.pallas_call(
        _corr_body,
        grid=(R // (TQ * K),),
        in_specs=[
            pl.BlockSpec((TQ * K, 256), lambda i: (i, 0)),
            pl.BlockSpec((TQ, 256), lambda i: (i, 0)),
            pl.BlockSpec((256, GROUPS), lambda i: (0, 0)),
            pl.BlockSpec((TQ * K, 16), lambda i: (i, 0)),
        ],
        out_specs=pl.BlockSpec((TQ * K, GROUPS + 6), lambda i: (i, 0)),
        out_shape=jax.ShapeDtypeStruct((R, GROUPS + 6), jnp.float32),
    )(gath, targets_flat, pmask, xout)


def kernel(xyz, fvec, targets, coords_world_xyz):
    B, N, C = fvec.shape
    M = targets.shape[1]
    BM = B * M

    xyzT = jnp.transpose(xyz, (0, 2, 1))
    d2, bmins4 = _d2_and_blockmins(coords_world_xyz, xyzT)

    d2r = d2.reshape(BM * NB, BLK)
    xyzpad = jnp.concatenate(
        [xyz.reshape(B * N, 3),
         jnp.zeros((B * N, 13), jnp.float32)], axis=-1)
    coordpad = jnp.concatenate(
        [coords_world_xyz.reshape(BM, 3),
         jnp.zeros((BM, 13), jnp.float32)], axis=-1)
    fvecf = fvec.reshape(B * N, C)

    gath, xout = _sc_select_gather(bmins4, d2r, xyzpad, coordpad, fvecf, BM, N)

    pmask = jnp.asarray(
        (np.arange(256)[:, None] // (C // GROUPS)
         == np.arange(GROUPS)[None, :]).astype(np.float32))
    out = _corr(gath, targets.reshape(BM, C), pmask, xout)
    return out.reshape(B, M, K, GROUPS + 6)
